# baseline (speedup 1.0000x reference)
import jax
import jax.numpy as jnp
from jax import lax
from jax.experimental import pallas as pl
from jax.experimental.pallas import tpu as pltpu
from jax.experimental.pallas import tpu_sc as plsc

R = 512
C = 256
OUT = 7
SR = 2
LANES = 16
NC = 2
NS = 16
NW = NC * NS
RPT = R // NW
IDX_PER_ROI = OUT * OUT * SR * SR * 4
ROWS_PER_P = OUT * SR * SR * 4
NBINS = OUT * OUT

_SIZES = (256, 128, 64, 32)
_SCALES = (0.25, 0.125, 0.0625, 0.03125)


def _pool_body(t0, t1, t2, t3, boxesT, meta, out, coords, metav, idx_buf,
               w_buf, gbuf, out_buf, sem0, sem1):
    tables = (t0, t1, t2, t3)
    cid = lax.axis_index("c")
    sid = lax.axis_index("s")
    wid = sid * NC + cid
    base_r = wid * RPT

    for i in range(4):
        pltpu.sync_copy(boxesT.at[i, pl.ds(base_r, RPT)], coords.at[i])
    for i in range(2):
        pltpu.sync_copy(meta.at[i, pl.ds(base_r, RPT)], metav.at[i])

    x1 = coords[0, :]
    y1 = coords[1, :]
    x2 = coords[2, :]
    y2 = coords[3, :]
    lvl = metav[0, :]
    bat = metav[1, :]

    def sel(vals, dtype):
        v = jnp.full((LANES,), vals[3], dtype)
        for l in (2, 1, 0):
            v = jnp.where(lvl == l, jnp.full((LANES,), vals[l], dtype), v)
        return v

    scale = sel(_SCALES, jnp.float32)
    szf = sel([float(s) for s in _SIZES], jnp.float32)
    szi = sel(_SIZES, jnp.int32)
    base_row = bat * (szi * szi)

    x1s = x1 * scale
    y1s = y1 * scale
    x2s = x2 * scale
    y2s = y2 * scale
    roi_w = jnp.maximum(x2s - x1s, 1.0)
    roi_h = jnp.maximum(y2s - y1s, 1.0)
    bin_w = roi_w / float(OUT)
    bin_h = roi_h / float(OUT)

    lane = lax.iota(jnp.int32, LANES)
    lane784 = lane * IDX_PER_ROI
    lane49 = lane * NBINS

    def sample_body(s, carry):
        p = s // 28
        i = (s // 14) % 2
        q = (s // 2) % 7
        j = s % 2
        pf = p.astype(jnp.float32)
        fi = i.astype(jnp.float32)
        qf = q.astype(jnp.float32)
        fj = j.astype(jnp.float32)
        Y = y1s + pf * bin_h + (fi + 0.5) * bin_h / float(SR)
        X = x1s + qf * bin_w + (fj + 0.5) * bin_w / float(SR)
        vmask = (Y > -1.0) & (Y < szf) & (X > -1.0) & (X < szf)
        vfac = jnp.where(vmask, jnp.float32(0.25), jnp.float32(0.0))
        y = jnp.clip(Y, 0.0, szf - 1.0)
        x = jnp.clip(X, 0.0, szf - 1.0)
        yl = y.astype(jnp.int32)
        xl = x.astype(jnp.int32)
        ylf = yl.astype(jnp.float32)
        xlf = xl.astype(jnp.float32)
        yh = jnp.minimum(yl + 1, szi - 1)
        xh = jnp.minimum(xl + 1, szi - 1)
        ly = y - ylf
        lx = x - xlf
        hy = 1.0 - ly
        hx = 1.0 - lx
        rl = base_row + yl * szi
        rh = base_row + yh * szi
        rows = (rl + xl, rl + xh, rh + xl, rh + xh)
        wts = (hy * hx * vfac, hy * lx * vfac, ly * hx * vfac, ly * lx * vfac)
        wbase = (p * 7 + q) * 16 + i * 8 + j * 4
        for c in range(4):
            plsc.store_scatter(idx_buf, [lane784 + (s * 4 + c)], rows[c])
            plsc.store_scatter(w_buf, [lane784 + (wbase + c)], wts[c])
        return carry

    lax.fori_loop(0, OUT * OUT * SR * SR, sample_body, 0)

    sems = (sem0, sem1)

    lvl_vec = lvl

    def issue(rp, slot):
        r = rp // 7
        p = rp % 7
        off = r * IDX_PER_ROI + p * ROWS_PER_P
        lvl_r = jnp.max(jnp.where(lane == r, lvl_vec, 0))
        for l in range(4):
            @pl.when(lvl_r == l)
            def _(tbl=tables[l]):
                pltpu.async_copy(tbl.at[idx_buf.at[pl.ds(off, ROWS_PER_P)]],
                                 gbuf.at[slot], sems[slot])

    issue(jnp.int32(0), 0)
    issue(jnp.int32(1), 1)

    nrp = RPT * 7

    def outer(it, carry):
        for b in range(2):
            rp = it * 2 + b
            r = rp // 7
            p = rp % 7
            pltpu.make_async_copy(t0.at[pl.ds(0, ROWS_PER_P)],
                                  gbuf.at[b], sems[b]).wait()

            def qbody(q, qc):
                sbin = p * 7 + q
                wb = r * IDX_PER_ROI + sbin * 16
                wt = [plsc.load_gather(
                          w_buf, [jnp.broadcast_to(wb + t, (LANES,))])
                      for t in range(16)]
                wtb = [plsc.pack(w, w, format=plsc.PackFormat.INTERLEAVED)
                       for w in wt]
                q8 = q * 8
                for d in range(8):
                    acc_e = None
                    acc_o = None
                    for i in range(2):
                        for j in range(2):
                            for c in range(4):
                                tau = i * 8 + j * 4 + c
                                row = q8 + i * 56 + j * 4 + c
                                v = plsc.bitcast(
                                    gbuf[b, row, pl.ds(d * LANES, LANES)],
                                    jnp.bfloat16)
                                pe, po = plsc.unpack(
                                    wtb[tau] * v,
                                    format=plsc.PackFormat.INTERLEAVED)
                                acc_e = pe if acc_e is None else acc_e + pe
                                acc_o = po if acc_o is None else acc_o + po
                    base = lane * (2 * NBINS) + (d * (32 * NBINS) + sbin)
                    plsc.store_scatter(out_buf, [base], acc_e)
                    plsc.store_scatter(out_buf, [base + NBINS], acc_o)
                return qc

            lax.fori_loop(0, 7, qbody, 0)

            @pl.when(p == 6)
            def _():
                pltpu.sync_copy(out_buf, out.at[base_r + r])

            @pl.when(rp < nrp - 2)
            def _():
                issue(rp + 2, b)
        return carry

    lax.fori_loop(0, nrp // 2, outer, 0)


def _run(tabs, boxesT, meta):
    mesh = plsc.VectorSubcoreMesh(core_axis_name="c", subcore_axis_name="s",
                                  num_cores=NC, num_subcores=NS)
    return pl.kernel(
        _pool_body,
        out_type=jax.ShapeDtypeStruct((R, C * NBINS), jnp.float32),
        mesh=mesh,
        compiler_params=pltpu.CompilerParams(needs_layout_passes=False),
        scratch_types=[
            pltpu.VMEM((4, LANES), jnp.float32),
            pltpu.VMEM((2, LANES), jnp.int32),
            pltpu.VMEM((RPT * IDX_PER_ROI,), jnp.int32),
            pltpu.VMEM((RPT * IDX_PER_ROI,), jnp.float32),
            pltpu.VMEM((2, ROWS_PER_P, C // 2), jnp.int32),
            pltpu.VMEM((C * NBINS,), jnp.float32),
            pltpu.SemaphoreType.DMA,
            pltpu.SemaphoreType.DMA,
        ],
    )(*tabs, boxesT, meta)


@jax.jit
def kernel(feat0, feat1, feat2, feat3, boxes, batch_ids):
    feats = (feat0, feat1, feat2, feat3)
    tabs = [jax.lax.bitcast_convert_type(
                f.astype(jnp.bfloat16).transpose(0, 2, 3, 1)
                 .reshape(-1, C // 2, 2),
                jnp.int32)
            for f in feats]
    area = (boxes[:, 2] - boxes[:, 0]) * (boxes[:, 3] - boxes[:, 1])
    s = jnp.sqrt(area)
    tl = jnp.floor(4.0 + jnp.log2(s / 224.0 + 1e-6))
    tl = jnp.clip(tl, 2.0, 5.0)
    levels = tl.astype(jnp.int32) - 2
    meta = jnp.stack([levels, batch_ids.astype(jnp.int32)], axis=0)
    return _run(tabs, boxes.T, meta).reshape(R, C, OUT, OUT)


# speedup vs baseline: 1.3205x; 1.3205x over previous
import jax
import jax.numpy as jnp
from jax import lax
from jax.experimental import pallas as pl
from jax.experimental.pallas import tpu as pltpu
from jax.experimental.pallas import tpu_sc as plsc

R = 512
C = 256
OUT = 7
SR = 2
LANES = 16
NC = 2
NS = 16
NW = NC * NS
RPT = R // NW
IDX_PER_ROI = OUT * OUT * SR * SR * 4
ROWS_PER_P = OUT * SR * SR * 4
NBINS = OUT * OUT

_SIZES = (256, 128, 64, 32)
_SCALES = (0.25, 0.125, 0.0625, 0.03125)


def _pool_body(t0, t1, t2, t3, boxesT, meta, out, coords, metav, idx_buf,
               w_buf, gbuf, out_buf, sem0, sem1):
    tables = (t0, t1, t2, t3)
    cid = lax.axis_index("c")
    sid = lax.axis_index("s")
    wid = sid * NC + cid
    base_r = wid * RPT

    for i in range(4):
        pltpu.sync_copy(boxesT.at[i, pl.ds(base_r, RPT)], coords.at[i])
    for i in range(2):
        pltpu.sync_copy(meta.at[i, pl.ds(base_r, RPT)], metav.at[i])

    x1 = coords[0, :]
    y1 = coords[1, :]
    x2 = coords[2, :]
    y2 = coords[3, :]
    lvl = metav[0, :]
    bat = metav[1, :]

    def sel(vals, dtype):
        v = jnp.full((LANES,), vals[3], dtype)
        for l in (2, 1, 0):
            v = jnp.where(lvl == l, jnp.full((LANES,), vals[l], dtype), v)
        return v

    scale = sel(_SCALES, jnp.float32)
    szf = sel([float(s) for s in _SIZES], jnp.float32)
    szi = sel(_SIZES, jnp.int32)
    base_row = bat * (szi * szi)

    x1s = x1 * scale
    y1s = y1 * scale
    x2s = x2 * scale
    y2s = y2 * scale
    roi_w = jnp.maximum(x2s - x1s, 1.0)
    roi_h = jnp.maximum(y2s - y1s, 1.0)
    bin_w = roi_w / float(OUT)
    bin_h = roi_h / float(OUT)

    lane = lax.iota(jnp.int32, LANES)
    lane784 = lane * IDX_PER_ROI
    lane49 = lane * NBINS

    def sample_body(s, carry):
        p = s // 28
        i = (s // 14) % 2
        q = (s // 2) % 7
        j = s % 2
        pf = p.astype(jnp.float32)
        fi = i.astype(jnp.float32)
        qf = q.astype(jnp.float32)
        fj = j.astype(jnp.float32)
        Y = y1s + pf * bin_h + (fi + 0.5) * bin_h / float(SR)
        X = x1s + qf * bin_w + (fj + 0.5) * bin_w / float(SR)
        vmask = (Y > -1.0) & (Y < szf) & (X > -1.0) & (X < szf)
        vfac = jnp.where(vmask, jnp.float32(0.25), jnp.float32(0.0))
        y = jnp.clip(Y, 0.0, szf - 1.0)
        x = jnp.clip(X, 0.0, szf - 1.0)
        yl = y.astype(jnp.int32)
        xl = x.astype(jnp.int32)
        ylf = yl.astype(jnp.float32)
        xlf = xl.astype(jnp.float32)
        yh = jnp.minimum(yl + 1, szi - 1)
        xh = jnp.minimum(xl + 1, szi - 1)
        ly = y - ylf
        lx = x - xlf
        hy = 1.0 - ly
        hx = 1.0 - lx
        rl = base_row + yl * szi
        rh = base_row + yh * szi
        rows = (rl + xl, rl + xh, rh + xl, rh + xh)
        wts = (hy * hx * vfac, hy * lx * vfac, ly * hx * vfac, ly * lx * vfac)
        wbase = (p * 7 + q) * 16 + i * 8 + j * 4
        for c in range(4):
            plsc.store_scatter(idx_buf, [lane784 + (s * 4 + c)], rows[c])
            plsc.store_scatter(w_buf, [lane784 + (wbase + c)], wts[c])
        return carry

    lax.fori_loop(0, OUT * OUT * SR * SR, sample_body, 0)

    sems = (sem0, sem1)

    lvl_vec = lvl

    def issue(rp, slot):
        r = rp // 7
        p = rp % 7
        off = r * IDX_PER_ROI + p * ROWS_PER_P
        lvl_r = jnp.max(jnp.where(lane == r, lvl_vec, 0))
        for l in range(4):
            @pl.when(lvl_r == l)
            def _(tbl=tables[l]):
                pltpu.async_copy(tbl.at[idx_buf.at[pl.ds(off, ROWS_PER_P)]],
                                 gbuf.at[slot], sems[slot])

    issue(jnp.int32(0), 0)
    issue(jnp.int32(1), 1)

    nrp = RPT * 7

    def outer(it, carry):
        for b in range(2):
            rp = it * 2 + b
            r = rp // 7
            p = rp % 7
            pltpu.make_async_copy(t0.at[pl.ds(0, ROWS_PER_P)],
                                  gbuf.at[b], sems[b]).wait()

            def qbody(q, qc):
                sbin = p * 7 + q
                wb = r * IDX_PER_ROI + sbin * 16
                wt = [plsc.load_gather(
                          w_buf, [jnp.broadcast_to(wb + t, (LANES,))])
                      for t in range(16)]
                wtb = [plsc.pack(w, w, format=plsc.PackFormat.INTERLEAVED)
                       for w in wt]
                q8 = q * 8
                for d in range(8):
                    acc_e = None
                    acc_o = None
                    for i in range(2):
                        for j in range(2):
                            for c in range(4):
                                tau = i * 8 + j * 4 + c
                                row = q8 + i * 56 + j * 4 + c
                                v = plsc.bitcast(
                                    gbuf[b, row, pl.ds(d * LANES, LANES)],
                                    jnp.bfloat16)
                                pe, po = plsc.unpack(
                                    wtb[tau] * v,
                                    format=plsc.PackFormat.INTERLEAVED)
                                acc_e = pe if acc_e is None else acc_e + pe
                                acc_o = po if acc_o is None else acc_o + po
                    base = lane * (2 * NBINS) + (d * (32 * NBINS) + sbin)
                    plsc.store_scatter(out_buf, [base], acc_e)
                    plsc.store_scatter(out_buf, [base + NBINS], acc_o)
                return qc

            lax.fori_loop(0, 7, qbody, 0)

            @pl.when(p == 6)
            def _():
                pltpu.sync_copy(out_buf, out.at[base_r + r])

            @pl.when(rp < nrp - 2)
            def _():
                issue(rp + 2, b)
        return carry

    lax.fori_loop(0, nrp // 2, outer, 0)


def _run(tabs, boxesT, meta):
    mesh = plsc.VectorSubcoreMesh(core_axis_name="c", subcore_axis_name="s",
                                  num_cores=NC, num_subcores=NS)
    return pl.kernel(
        _pool_body,
        out_type=jax.ShapeDtypeStruct((R, C * NBINS), jnp.float32),
        mesh=mesh,
        compiler_params=pltpu.CompilerParams(needs_layout_passes=False),
        scratch_types=[
            pltpu.VMEM((4, LANES), jnp.float32),
            pltpu.VMEM((2, LANES), jnp.int32),
            pltpu.VMEM((RPT * IDX_PER_ROI,), jnp.int32),
            pltpu.VMEM((RPT * IDX_PER_ROI,), jnp.float32),
            pltpu.VMEM((2, ROWS_PER_P, C // 2), jnp.int32),
            pltpu.VMEM((C * NBINS,), jnp.float32),
            pltpu.SemaphoreType.DMA,
            pltpu.SemaphoreType.DMA,
        ],
    )(*tabs, boxesT, meta)


@jax.jit
def kernel(feat0, feat1, feat2, feat3, boxes, batch_ids):
    feats = (feat0, feat1, feat2, feat3)
    def pack_level(f):
        u = jax.lax.bitcast_convert_type(f.astype(jnp.bfloat16), jnp.uint16)
        pk = (u[:, 0::2].astype(jnp.uint32)
              | (u[:, 1::2].astype(jnp.uint32) << 16))
        pk = jax.lax.bitcast_convert_type(pk, jnp.int32)
        return pk.transpose(0, 2, 3, 1).reshape(-1, C // 2)

    tabs = [pack_level(f) for f in feats]
    area = (boxes[:, 2] - boxes[:, 0]) * (boxes[:, 3] - boxes[:, 1])
    s = jnp.sqrt(area)
    tl = jnp.floor(4.0 + jnp.log2(s / 224.0 + 1e-6))
    tl = jnp.clip(tl, 2.0, 5.0)
    levels = tl.astype(jnp.int32) - 2
    meta = jnp.stack([levels, batch_ids.astype(jnp.int32)], axis=0)
    return _run(tabs, boxes.T, meta).reshape(R, C, OUT, OUT)


# speedup vs baseline: 2.2242x; 1.6844x over previous
import jax
import jax.numpy as jnp
from jax import lax
from jax.experimental import pallas as pl
from jax.experimental.pallas import tpu as pltpu
from jax.experimental.pallas import tpu_sc as plsc

R = 512
C = 256
OUT = 7
SR = 2
LANES = 16
NC = 2
NS = 16
NW = NC * NS
RPT = R // NW
IDX_PER_ROI = OUT * OUT * SR * SR * 4
ROWS_PER_P = OUT * SR * SR * 4
NBINS = OUT * OUT

_SIZES = (256, 128, 64, 32)
_SCALES = (0.25, 0.125, 0.0625, 0.03125)


def _pool_body(t0, t1, t2, t3, boxesT, meta, out, coords, metav, idx_buf,
               w_buf, gbuf, out_buf, sem0, sem1):
    tables = (t0, t1, t2, t3)
    cid = lax.axis_index("c")
    sid = lax.axis_index("s")
    wid = sid * NC + cid
    base_r = wid * RPT

    for i in range(4):
        pltpu.sync_copy(boxesT.at[i, pl.ds(base_r, RPT)], coords.at[i])
    for i in range(2):
        pltpu.sync_copy(meta.at[i, pl.ds(base_r, RPT)], metav.at[i])

    x1 = coords[0, :]
    y1 = coords[1, :]
    x2 = coords[2, :]
    y2 = coords[3, :]
    lvl = metav[0, :]
    bat = metav[1, :]

    def sel(vals, dtype):
        v = jnp.full((LANES,), vals[3], dtype)
        for l in (2, 1, 0):
            v = jnp.where(lvl == l, jnp.full((LANES,), vals[l], dtype), v)
        return v

    scale = sel(_SCALES, jnp.float32)
    szf = sel([float(s) for s in _SIZES], jnp.float32)
    szi = sel(_SIZES, jnp.int32)
    base_row = bat * (szi * szi)

    x1s = x1 * scale
    y1s = y1 * scale
    x2s = x2 * scale
    y2s = y2 * scale
    roi_w = jnp.maximum(x2s - x1s, 1.0)
    roi_h = jnp.maximum(y2s - y1s, 1.0)
    bin_w = roi_w / float(OUT)
    bin_h = roi_h / float(OUT)

    lane = lax.iota(jnp.int32, LANES)
    lane784 = lane * IDX_PER_ROI
    lane49 = lane * NBINS

    def sample_body(s, carry):
        p = s // 28
        i = (s // 14) % 2
        q = (s // 2) % 7
        j = s % 2
        pf = p.astype(jnp.float32)
        fi = i.astype(jnp.float32)
        qf = q.astype(jnp.float32)
        fj = j.astype(jnp.float32)
        Y = y1s + pf * bin_h + (fi + 0.5) * bin_h / float(SR)
        X = x1s + qf * bin_w + (fj + 0.5) * bin_w / float(SR)
        vmask = (Y > -1.0) & (Y < szf) & (X > -1.0) & (X < szf)
        vfac = jnp.where(vmask, jnp.float32(0.25), jnp.float32(0.0))
        y = jnp.clip(Y, 0.0, szf - 1.0)
        x = jnp.clip(X, 0.0, szf - 1.0)
        yl = y.astype(jnp.int32)
        xl = x.astype(jnp.int32)
        ylf = yl.astype(jnp.float32)
        xlf = xl.astype(jnp.float32)
        yh = jnp.minimum(yl + 1, szi - 1)
        xh = jnp.minimum(xl + 1, szi - 1)
        ly = y - ylf
        lx = x - xlf
        hy = 1.0 - ly
        hx = 1.0 - lx
        rl = base_row + yl * szi
        rh = base_row + yh * szi
        rows = (rl + xl, rl + xh, rh + xl, rh + xh)
        wts = (hy * hx * vfac, hy * lx * vfac, ly * hx * vfac, ly * lx * vfac)
        wbase = (p * 7 + q) * 16 + i * 8 + j * 4
        for c in range(4):
            plsc.store_scatter(idx_buf, [lane784 + (s * 4 + c)], rows[c])
            plsc.store_scatter(w_buf, [lane784 + (wbase + c)], wts[c])
        return carry

    lax.fori_loop(0, OUT * OUT * SR * SR, sample_body, 0)

    sems = (sem0, sem1)

    lvl_vec = lvl

    def issue(rp, slot):
        r = rp // 7
        p = rp % 7
        off = r * IDX_PER_ROI + p * ROWS_PER_P
        lvl_r = jnp.max(jnp.where(lane == r, lvl_vec, 0))
        for l in range(4):
            @pl.when(lvl_r == l)
            def _(tbl=tables[l]):
                pltpu.async_copy(tbl.at[idx_buf.at[pl.ds(off, ROWS_PER_P)]],
                                 gbuf.at[slot], sems[slot])

    issue(jnp.int32(0), 0)
    issue(jnp.int32(1), 1)

    nrp = RPT * 7

    def outer(it, carry):
        for b in range(2):
            rp = it * 2 + b
            r = rp // 7
            p = rp % 7
            pltpu.make_async_copy(t0.at[pl.ds(0, ROWS_PER_P)],
                                  gbuf.at[b], sems[b]).wait()

            def qbody(q, qc):
                sbin = p * 7 + q
                wb = r * IDX_PER_ROI + sbin * 16
                wt = [plsc.load_gather(
                          w_buf, [jnp.broadcast_to(wb + t, (LANES,))])
                      for t in range(16)]
                wtb = [plsc.pack(w, w, format=plsc.PackFormat.INTERLEAVED)
                       for w in wt]
                q8 = q * 8
                for d in range(8):
                    acc_e = None
                    acc_o = None
                    for i in range(2):
                        for j in range(2):
                            for c in range(4):
                                tau = i * 8 + j * 4 + c
                                row = q8 + i * 56 + j * 4 + c
                                v = plsc.bitcast(
                                    gbuf[b, row, pl.ds(d * LANES, LANES)],
                                    jnp.bfloat16)
                                pe, po = plsc.unpack(
                                    wtb[tau] * v,
                                    format=plsc.PackFormat.INTERLEAVED)
                                acc_e = pe if acc_e is None else acc_e + pe
                                acc_o = po if acc_o is None else acc_o + po
                    base = lane49 + (d * (LANES * NBINS) + sbin)
                    plsc.store_scatter(out_buf, [base], acc_e)
                    plsc.store_scatter(out_buf, [base + (C // 2) * NBINS],
                                       acc_o)
                return qc

            lax.fori_loop(0, 7, qbody, 0)

            @pl.when(p == 6)
            def _():
                pltpu.sync_copy(out_buf, out.at[base_r + r])

            @pl.when(rp < nrp - 2)
            def _():
                issue(rp + 2, b)
        return carry

    lax.fori_loop(0, nrp // 2, outer, 0)


def _run(tabs, boxesT, meta):
    mesh = plsc.VectorSubcoreMesh(core_axis_name="c", subcore_axis_name="s",
                                  num_cores=NC, num_subcores=NS)
    return pl.kernel(
        _pool_body,
        out_type=jax.ShapeDtypeStruct((R, C * NBINS), jnp.float32),
        mesh=mesh,
        compiler_params=pltpu.CompilerParams(needs_layout_passes=False),
        scratch_types=[
            pltpu.VMEM((4, LANES), jnp.float32),
            pltpu.VMEM((2, LANES), jnp.int32),
            pltpu.VMEM((RPT * IDX_PER_ROI,), jnp.int32),
            pltpu.VMEM((RPT * IDX_PER_ROI,), jnp.float32),
            pltpu.VMEM((2, ROWS_PER_P, C // 2), jnp.int32),
            pltpu.VMEM((C * NBINS,), jnp.float32),
            pltpu.SemaphoreType.DMA,
            pltpu.SemaphoreType.DMA,
        ],
    )(*tabs, boxesT, meta)


@jax.jit
def kernel(feat0, feat1, feat2, feat3, boxes, batch_ids):
    feats = (feat0, feat1, feat2, feat3)
    def pack_level(f):
        u = jax.lax.bitcast_convert_type(f.astype(jnp.bfloat16), jnp.uint16)
        pk = (u[:, :C // 2].astype(jnp.uint32)
              | (u[:, C // 2:].astype(jnp.uint32) << 16))
        pk = jax.lax.bitcast_convert_type(pk, jnp.int32)
        return pk.transpose(0, 2, 3, 1).reshape(-1, C // 2)

    tabs = [pack_level(f) for f in feats]
    area = (boxes[:, 2] - boxes[:, 0]) * (boxes[:, 3] - boxes[:, 1])
    s = jnp.sqrt(area)
    tl = jnp.floor(4.0 + jnp.log2(s / 224.0 + 1e-6))
    tl = jnp.clip(tl, 2.0, 5.0)
    levels = tl.astype(jnp.int32) - 2
    meta = jnp.stack([levels, batch_ids.astype(jnp.int32)], axis=0)
    return _run(tabs, boxes.T, meta).reshape(R, C, OUT, OUT)
